# baseline (device time: 273141 ns/iter reference)
import jax
import jax.numpy as jnp
from jax import lax
from jax.experimental import pallas as pl
from jax.experimental.pallas import tpu as pltpu

B = 4
S = 1024
K = 16 * 128
N = 4096
HALF = 512
SB = 512
NSB = HALF // SB
NCH = B * NSB
WB = 128
NWB = K // WB


def kernel(O, Wo):
    O2 = O.reshape(B, S, K)

    def body(o_hbm, wo_hbm, out_hbm, wo16, wo_stage, o_stage,
             send_buf, recv_buf, my_buf,
             wload_sems, load_sems, store_sems, send_sems, recv_sems,
             credit_sem):
        my_z = lax.axis_index("z")
        peer = (lax.axis_index("x"), lax.axis_index("y"), 1 - my_z)
        my_s0 = my_z * HALF
        peer_s0 = (1 - my_z) * HALF

        barrier = pltpu.get_barrier_semaphore()
        pl.semaphore_signal(barrier, inc=1, device_id=peer,
                            device_id_type=pl.DeviceIdType.MESH)
        pl.semaphore_wait(barrier, 1)

        def wo_load(r):
            return pltpu.make_async_copy(
                wo_hbm.at[pl.ds(r * WB, WB), :], wo_stage.at[r % 2],
                wload_sems.at[r % 2])

        wo_load(0).start()
        for r in range(NWB):
            if r + 1 < NWB:
                wo_load(r + 1).start()
            wo_load(r).wait()
            wo16[pl.ds(r * WB, WB), :] = wo_stage[r % 2].astype(jnp.bfloat16)

        def exchange(slot):
            return pltpu.make_async_remote_copy(
                src_ref=send_buf.at[slot], dst_ref=recv_buf.at[slot],
                send_sem=send_sems.at[slot], recv_sem=recv_sems.at[slot],
                device_id=peer, device_id_type=pl.DeviceIdType.MESH)

        def store(slot, b, sb):
            return pltpu.make_async_copy(
                my_buf.at[slot], out_hbm.at[b, pl.ds(sb * SB, SB), :],
                store_sems.at[slot])

        def step(k, _):
            b = k // NSB
            sb = k % NSB
            slot = lax.rem(k, 2)
            pslot = lax.rem(k + 1, 2)

            ld = pltpu.make_async_copy(
                o_hbm.at[b, pl.ds(peer_s0 + sb * SB, SB), :],
                o_stage.at[0], load_sems.at[0])
            ld.start()
            ld2 = pltpu.make_async_copy(
                o_hbm.at[b, pl.ds(my_s0 + sb * SB, SB), :],
                o_stage.at[1], load_sems.at[1])
            ld2.start()
            ld.wait()

            @pl.when(k >= 2)
            def _():
                exchange(slot).wait_send()

            send_buf[slot] = jnp.dot(
                o_stage[0].astype(jnp.bfloat16), wo16[...],
                preferred_element_type=jnp.float32).astype(jnp.bfloat16)

            @pl.when(k >= 2)
            def _():
                pl.semaphore_wait(credit_sem, 1)
            exchange(slot).start()

            ld2.wait()

            @pl.when(k >= 2)
            def _():
                store(slot, (k - 2) // NSB, (k - 2) % NSB).wait()

            my_buf[slot] = jnp.dot(
                o_stage[1].astype(jnp.bfloat16), wo16[...],
                preferred_element_type=jnp.float32)

            @pl.when(k >= 1)
            def _():
                exchange(pslot).wait_recv()
                my_buf[pslot] = (my_buf[pslot]
                                 + recv_buf[pslot].astype(jnp.float32))

            @pl.when(jnp.logical_and(k >= 1, k <= NCH - 2))
            def _():
                pl.semaphore_signal(credit_sem, inc=1, device_id=peer,
                                    device_id_type=pl.DeviceIdType.MESH)

            @pl.when(k >= 1)
            def _():
                store(pslot, (k - 1) // NSB, (k - 1) % NSB).start()
            return 0

        lax.fori_loop(0, NCH, step, 0)

        last = NCH - 1
        lslot = last % 2
        exchange(lslot).wait_recv()
        my_buf[lslot] = my_buf[lslot] + recv_buf[lslot].astype(jnp.float32)
        store(lslot, last // NSB, last % NSB).start()
        for k in (NCH - 2, NCH - 1):
            exchange(k % 2).wait_send()
            store(k % 2, k // NSB, k % NSB).wait()

    return pl.pallas_call(
        body,
        out_shape=jax.ShapeDtypeStruct((B, HALF, N), jnp.float32),
        in_specs=[
            pl.BlockSpec(memory_space=pl.ANY),
            pl.BlockSpec(memory_space=pl.ANY),
        ],
        out_specs=pl.BlockSpec(memory_space=pl.ANY),
        scratch_shapes=[
            pltpu.VMEM((K, N), jnp.bfloat16),
            pltpu.VMEM((2, WB, N), jnp.float32),
            pltpu.VMEM((2, SB, K), jnp.float32),
            pltpu.VMEM((2, SB, N), jnp.bfloat16),
            pltpu.VMEM((2, SB, N), jnp.bfloat16),
            pltpu.VMEM((2, SB, N), jnp.float32),
            pltpu.SemaphoreType.DMA((2,)),
            pltpu.SemaphoreType.DMA((2,)),
            pltpu.SemaphoreType.DMA((2,)),
            pltpu.SemaphoreType.DMA((2,)),
            pltpu.SemaphoreType.DMA((2,)),
            pltpu.SemaphoreType.REGULAR,
        ],
        compiler_params=pltpu.CompilerParams(
            collective_id=0,
            vmem_limit_bytes=64 * 1024 * 1024,
        ),
    )(O2, Wo)


# device time: 266171 ns/iter; 1.0262x vs baseline; 1.0262x over previous
import jax
import jax.numpy as jnp
from jax import lax
from jax.experimental import pallas as pl
from jax.experimental.pallas import tpu as pltpu

B = 4
S = 1024
K = 16 * 128
N = 4096
HALF = 512
SB = 256
NSB = HALF // SB
NCH = B * NSB
WB = 256
NWB = K // WB


def kernel(O, Wo):
    O2 = O.reshape(B, S, K)

    def body(o_hbm, wo_hbm, out_hbm, wo16, wo_stage, o_peer, o_my,
             send_buf, recv_buf, my_buf,
             wload_sems, peer_sem, my_sems, store_sems, send_sems, recv_sems,
             credit_sem):
        my_z = lax.axis_index("z")
        peer = (lax.axis_index("x"), lax.axis_index("y"), 1 - my_z)
        my_s0 = my_z * HALF
        peer_s0 = (1 - my_z) * HALF

        barrier = pltpu.get_barrier_semaphore()
        pl.semaphore_signal(barrier, inc=1, device_id=peer,
                            device_id_type=pl.DeviceIdType.MESH)
        pl.semaphore_wait(barrier, 1)

        def wo_load(r):
            return pltpu.make_async_copy(
                wo_hbm.at[pl.ds(r * WB, WB), :], wo_stage.at[r % 2],
                wload_sems.at[r % 2])

        wo_load(0).start()
        for r in range(NWB):
            if r + 1 < NWB:
                wo_load(r + 1).start()
            wo_load(r).wait()
            wo16[pl.ds(r * WB, WB), :] = wo_stage[r % 2].astype(jnp.bfloat16)

        def exchange(slot):
            return pltpu.make_async_remote_copy(
                src_ref=send_buf.at[slot], dst_ref=recv_buf.at[slot],
                send_sem=send_sems.at[slot], recv_sem=recv_sems.at[slot],
                device_id=peer, device_id_type=pl.DeviceIdType.MESH)

        def my_load(j):
            b, sb = j // NSB, j % NSB
            return pltpu.make_async_copy(
                o_hbm.at[b, pl.ds(my_s0 + sb * SB, SB), :],
                o_my.at[lax.rem(j, 2)], my_sems.at[lax.rem(j, 2)])

        def store(j):
            b, sb = j // NSB, j % NSB
            return pltpu.make_async_copy(
                my_buf.at[lax.rem(j, 2)],
                out_hbm.at[b, pl.ds(sb * SB, SB), :],
                store_sems.at[lax.rem(j, 2)])

        def consume(j):
            pslot = lax.rem(j, 2)
            my_load(j).wait()
            exchange(pslot).wait_recv()
            my_buf[pslot] = (
                jnp.dot(o_my[pslot].astype(jnp.bfloat16), wo16[...],
                        preferred_element_type=jnp.float32)
                + recv_buf[pslot].astype(jnp.float32))

        def step(k, _):
            b = k // NSB
            sb = k % NSB
            slot = lax.rem(k, 2)

            ld = pltpu.make_async_copy(
                o_hbm.at[b, pl.ds(peer_s0 + sb * SB, SB), :],
                o_peer, peer_sem)
            ld.start()
            my_load(k).start()
            ld.wait()

            @pl.when(k >= 2)
            def _():
                exchange(slot).wait_send()

            send_buf[slot] = jnp.dot(
                o_peer[...].astype(jnp.bfloat16), wo16[...],
                preferred_element_type=jnp.float32).astype(jnp.bfloat16)

            @pl.when(k >= 2)
            def _():
                pl.semaphore_wait(credit_sem, 1)
            exchange(slot).start()

            @pl.when(k >= 3)
            def _():
                store(k - 3).wait()

            @pl.when(k >= 1)
            def _():
                consume(k - 1)

            @pl.when(jnp.logical_and(k >= 1, k <= NCH - 2))
            def _():
                pl.semaphore_signal(credit_sem, inc=1, device_id=peer,
                                    device_id_type=pl.DeviceIdType.MESH)

            @pl.when(k >= 1)
            def _():
                store(k - 1).start()
            return 0

        lax.fori_loop(0, NCH, step, 0)

        store(NCH - 3).wait()
        consume(NCH - 1)
        store(NCH - 1).start()
        exchange(0).wait_send()
        exchange(1).wait_send()
        store(NCH - 2).wait()
        store(NCH - 1).wait()

    return pl.pallas_call(
        body,
        out_shape=jax.ShapeDtypeStruct((B, HALF, N), jnp.float32),
        in_specs=[
            pl.BlockSpec(memory_space=pl.ANY),
            pl.BlockSpec(memory_space=pl.ANY),
        ],
        out_specs=pl.BlockSpec(memory_space=pl.ANY),
        scratch_shapes=[
            pltpu.VMEM((K, N), jnp.bfloat16),
            pltpu.VMEM((2, WB, N), jnp.float32),
            pltpu.VMEM((SB, K), jnp.float32),
            pltpu.VMEM((2, SB, K), jnp.float32),
            pltpu.VMEM((2, SB, N), jnp.bfloat16),
            pltpu.VMEM((2, SB, N), jnp.bfloat16),
            pltpu.VMEM((2, SB, N), jnp.float32),
            pltpu.SemaphoreType.DMA((2,)),
            pltpu.SemaphoreType.DMA,
            pltpu.SemaphoreType.DMA((2,)),
            pltpu.SemaphoreType.DMA((2,)),
            pltpu.SemaphoreType.DMA((2,)),
            pltpu.SemaphoreType.DMA((2,)),
            pltpu.SemaphoreType.REGULAR,
        ],
        compiler_params=pltpu.CompilerParams(
            collective_id=0,
            vmem_limit_bytes=64 * 1024 * 1024,
        ),
    )(O2, Wo)


# device time: 262194 ns/iter; 1.0418x vs baseline; 1.0152x over previous
import jax
import jax.numpy as jnp
from jax import lax
from jax.experimental import pallas as pl
from jax.experimental.pallas import tpu as pltpu

B = 4
S = 1024
K = 16 * 128
N = 4096
HALF = 512
SB = 256
NSB = HALF // SB
NCH = B * NSB
WB = 256
NWB = K // WB


def kernel(O, Wo):
    O2 = O.reshape(B, S, K)

    def body(o_hbm, wo_hbm, out_hbm, wo16, wo_stage, o_stage,
             send_buf, recv_buf, my_buf,
             wload_sems, load_sems, store_sems, send_sems, recv_sems,
             credit_sem):
        my_z = lax.axis_index("z")
        peer = (lax.axis_index("x"), lax.axis_index("y"), 1 - my_z)
        my_s0 = my_z * HALF
        peer_s0 = (1 - my_z) * HALF

        barrier = pltpu.get_barrier_semaphore()
        pl.semaphore_signal(barrier, inc=1, device_id=peer,
                            device_id_type=pl.DeviceIdType.MESH)
        pl.semaphore_wait(barrier, 1)

        def wo_load(r):
            return pltpu.make_async_copy(
                wo_hbm.at[pl.ds(r * WB, WB), :], wo_stage.at[r % 2],
                wload_sems.at[r % 2])

        wo_load(0).start()
        for r in range(NWB):
            if r + 1 < NWB:
                wo_load(r + 1).start()
            wo_load(r).wait()
            wo16[pl.ds(r * WB, WB), :] = wo_stage[r % 2].astype(jnp.bfloat16)

        def exchange(slot):
            return pltpu.make_async_remote_copy(
                src_ref=send_buf.at[slot], dst_ref=recv_buf.at[slot],
                send_sem=send_sems.at[slot], recv_sem=recv_sems.at[slot],
                device_id=peer, device_id_type=pl.DeviceIdType.MESH)

        def store(slot, b, sb):
            return pltpu.make_async_copy(
                my_buf.at[slot], out_hbm.at[b, pl.ds(sb * SB, SB), :],
                store_sems.at[slot])

        def step(k, _):
            b = k // NSB
            sb = k % NSB
            slot = lax.rem(k, 2)
            pslot = lax.rem(k + 1, 2)

            ld = pltpu.make_async_copy(
                o_hbm.at[b, pl.ds(peer_s0 + sb * SB, SB), :],
                o_stage.at[0], load_sems.at[0])
            ld.start()
            ld2 = pltpu.make_async_copy(
                o_hbm.at[b, pl.ds(my_s0 + sb * SB, SB), :],
                o_stage.at[1], load_sems.at[1])
            ld2.start()
            ld.wait()

            @pl.when(k >= 2)
            def _():
                exchange(slot).wait_send()

            send_buf[slot] = jnp.dot(
                o_stage[0].astype(jnp.bfloat16), wo16[...],
                preferred_element_type=jnp.float32).astype(jnp.bfloat16)

            @pl.when(k >= 2)
            def _():
                pl.semaphore_wait(credit_sem, 1)
            exchange(slot).start()

            ld2.wait()

            @pl.when(k >= 2)
            def _():
                store(slot, (k - 2) // NSB, (k - 2) % NSB).wait()

            my_buf[slot] = jnp.dot(
                o_stage[1].astype(jnp.bfloat16), wo16[...],
                preferred_element_type=jnp.float32)

            @pl.when(k >= 1)
            def _():
                exchange(pslot).wait_recv()
                my_buf[pslot] = (my_buf[pslot]
                                 + recv_buf[pslot].astype(jnp.float32))

            @pl.when(jnp.logical_and(k >= 1, k <= NCH - 2))
            def _():
                pl.semaphore_signal(credit_sem, inc=1, device_id=peer,
                                    device_id_type=pl.DeviceIdType.MESH)

            @pl.when(k >= 1)
            def _():
                store(pslot, (k - 1) // NSB, (k - 1) % NSB).start()
            return 0

        lax.fori_loop(0, NCH, step, 0)

        last = NCH - 1
        lslot = last % 2
        exchange(lslot).wait_recv()
        my_buf[lslot] = my_buf[lslot] + recv_buf[lslot].astype(jnp.float32)
        store(lslot, last // NSB, last % NSB).start()
        for k in (NCH - 2, NCH - 1):
            exchange(k % 2).wait_send()
            store(k % 2, k // NSB, k % NSB).wait()

    return pl.pallas_call(
        body,
        out_shape=jax.ShapeDtypeStruct((B, HALF, N), jnp.float32),
        in_specs=[
            pl.BlockSpec(memory_space=pl.ANY),
            pl.BlockSpec(memory_space=pl.ANY),
        ],
        out_specs=pl.BlockSpec(memory_space=pl.ANY),
        scratch_shapes=[
            pltpu.VMEM((K, N), jnp.bfloat16),
            pltpu.VMEM((2, WB, N), jnp.float32),
            pltpu.VMEM((2, SB, K), jnp.float32),
            pltpu.VMEM((2, SB, N), jnp.bfloat16),
            pltpu.VMEM((2, SB, N), jnp.bfloat16),
            pltpu.VMEM((2, SB, N), jnp.float32),
            pltpu.SemaphoreType.DMA((2,)),
            pltpu.SemaphoreType.DMA((2,)),
            pltpu.SemaphoreType.DMA((2,)),
            pltpu.SemaphoreType.DMA((2,)),
            pltpu.SemaphoreType.DMA((2,)),
            pltpu.SemaphoreType.REGULAR,
        ],
        compiler_params=pltpu.CompilerParams(
            collective_id=0,
            vmem_limit_bytes=64 * 1024 * 1024,
        ),
    )(O2, Wo)
